# Initial kernel scaffold; baseline (speedup 1.0000x reference)
import jax
import jax.numpy as jnp
from jax.experimental import pallas as pl


def kernel(x, W0, W1, W2, W3, W4, W5, W6, W7, W8):
    raise NotImplementedError("write your pallas kernel here")


# speedup vs baseline: 12.3480x; 12.3480x over previous
import functools

import jax
import jax.numpy as jnp
from jax.experimental import pallas as pl

_DIMS = (119, 4, 12, 14, 17, 8, 14, 2, 10)
_OFFS = tuple(int(sum(_DIMS[:i])) for i in range(len(_DIMS)))
_VOCAB = sum(_DIMS)
_VPAD = 256
_EMB = 128
_B = 2000


def _body(x_ref, w_ref, o_ref):
    xb = x_ref[0]
    iota = jax.lax.broadcasted_iota(jnp.int32, (_B, _VPAD), 1)
    mh = jnp.zeros((_B, _VPAD), jnp.float32)
    for i in range(len(_DIMS)):
        idx = xb[i, :][:, None] + _OFFS[i]
        mh = mh + (iota == idx).astype(jnp.float32)
    o_ref[...] = jnp.dot(mh, w_ref[...], preferred_element_type=jnp.float32)


@functools.partial(jax.jit, static_argnames=("interpret",))
def _run(x, Ws, interpret=False):
    n = x.shape[0]
    nb = n // _B
    x3 = x.reshape(nb, _B, 9).transpose(0, 2, 1)
    wcat = jnp.concatenate(Ws, axis=0)
    wcat = jnp.pad(wcat, ((0, _VPAD - _VOCAB), (0, 0)))
    return pl.pallas_call(
        _body,
        grid=(nb,),
        in_specs=[
            pl.BlockSpec((1, 9, _B), lambda i: (i, 0, 0)),
            pl.BlockSpec((_VPAD, _EMB), lambda i: (0, 0)),
        ],
        out_specs=pl.BlockSpec((_B, _EMB), lambda i: (i, 0)),
        out_shape=jax.ShapeDtypeStruct((n, _EMB), jnp.float32),
        interpret=interpret,
    )(x3, wcat)


def kernel(x, W0, W1, W2, W3, W4, W5, W6, W7, W8):
    return _run(x, (W0, W1, W2, W3, W4, W5, W6, W7, W8))
